# baseline (speedup 1.0000x reference)
import jax
import jax.numpy as jnp
import numpy as np
from jax import lax
from jax.experimental import pallas as pl
from jax.experimental.pallas import tpu as pltpu
from jax.experimental.pallas import tpu_sc as plsc

_D = 1024
_NL = 2
_NSTEP = 3 * _NL
_NW = 32
_G = 16


def _trig_body(t_ref, c_ref, s_ref):
    t = t_ref[...]
    c_ref[...] = jnp.cos(t)
    s_ref[...] = jnp.sin(t)


def _pair_trig_tables(local_angles, ent_angles):
    rows = []
    for l in range(_NL):
        rows.append(local_angles[l, 0::2])
        rows.append(local_angles[l, 1::2])
        rows.append(ent_angles[l])
    theta = jnp.stack(rows)
    return pl.pallas_call(
        _trig_body,
        out_shape=(
            jax.ShapeDtypeStruct(theta.shape, theta.dtype),
            jax.ShapeDtypeStruct(theta.shape, theta.dtype),
        ),
    )(theta)


def _tc_body(theta_ref, x_ref, o_ref):
    v = x_ref[...]
    lane = lax.broadcasted_iota(jnp.int32, (1, _D), 1)
    parity = lane % 2
    for k in range(_NSTEP):
        p = 1 if (k % 3 == 1) else 0
        t = theta_ref[k, :].reshape(1, _D)
        c = jnp.cos(t)
        s = jnp.sin(t)
        is_lo = parity == p
        s_signed = jnp.where(is_lo, s, -s)
        partner = jnp.where(is_lo, jnp.roll(v, -1, axis=1), jnp.roll(v, 1, axis=1))
        v = c * v + s_signed * partner
    o_ref[...] = v


def _theta_table(local_angles, ent_angles):
    evenm = jnp.asarray((np.arange(_D) % 2) == 0)
    rows = []
    for l in range(_NL):
        a = local_angles[l]
        rows.append(jnp.where(evenm, a, jnp.roll(a, 1)))
        rows.append(jnp.where(~evenm, a, jnp.roll(a, 1)))
        rows.append(jnp.repeat(ent_angles[l], 2))
    return jnp.stack(rows)


def _band_body(theta_ref, w_ref):
    lane = lax.broadcasted_iota(jnp.int32, (1, _D), 1)
    parity = lane % 2
    row = lax.broadcasted_iota(jnp.int32, (16, _D), 0)
    b_mat = jnp.where(row == 8, 1.0, 0.0)
    for k in range(_NSTEP):
        p = 1 if (k % 3 == 1) else 0
        t = theta_ref[k, :].reshape(1, _D)
        c = jnp.cos(t)
        s = jnp.sin(t)
        is_lo = parity == p
        a = c
        b = jnp.where(is_lo, s, 0.0)
        d = jnp.where(is_lo, 0.0, -s)
        b_mat = (
            a * b_mat
            + b * jnp.roll(jnp.roll(b_mat, 1, axis=0), -1, axis=1)
            + d * jnp.roll(jnp.roll(b_mat, -1, axis=0), 1, axis=1)
        )
    w_ref[...] = b_mat


def _banded_main_body(w_ref, x_ref, o_ref):
    v = x_ref[...]
    acc = w_ref[8, :].reshape(1, _D) * v
    for r in list(range(2, 8)) + list(range(9, 15)):
        acc += w_ref[r, :].reshape(1, _D) * jnp.roll(v, 8 - r, axis=1)
    o_ref[...] = acc


def _tc_apply_banded(xf, local_angles, ent_angles, br=512):
    rows = xf.shape[0]
    theta = _theta_table(local_angles, ent_angles)
    w = pl.pallas_call(
        _band_body,
        out_shape=jax.ShapeDtypeStruct((16, _D), jnp.float32),
    )(theta)
    return pl.pallas_call(
        _banded_main_body,
        grid=(rows // br,),
        in_specs=[
            pl.BlockSpec((16, _D), lambda i: (0, 0)),
            pl.BlockSpec((br, _D), lambda i: (i, 0)),
        ],
        out_specs=pl.BlockSpec((br, _D), lambda i: (i, 0)),
        out_shape=jax.ShapeDtypeStruct((rows, _D), xf.dtype),
    )(w, xf)


def _mat_body(w_ref, m_ref):
    ri = lax.broadcasted_iota(jnp.int32, (_D, _D), 0)
    ci = lax.broadcasted_iota(jnp.int32, (_D, _D), 1)
    acc = jnp.zeros((_D, _D), jnp.float32)
    for r in range(2, 15):
        wcol = w_ref[r, :].reshape(1, _D)
        acc = jnp.where((ri - ci - (r - 8)) % _D == 0, wcol, acc)
    m_ref[...] = acc


def _mxu_main_body(m_ref, x_ref, o_ref):
    o_ref[...] = jax.lax.dot_general(
        x_ref[...], m_ref[...],
        (((1,), (0,)), ((), ())),
        preferred_element_type=jnp.float32,
    )


def _tc_apply_mxu(xf, local_angles, ent_angles, br=512):
    rows = xf.shape[0]
    theta = _theta_table(local_angles, ent_angles)
    w = pl.pallas_call(
        _band_body,
        out_shape=jax.ShapeDtypeStruct((16, _D), jnp.float32),
    )(theta)
    m = pl.pallas_call(
        _mat_body,
        out_shape=jax.ShapeDtypeStruct((_D, _D), jnp.float32),
    )(w)
    return pl.pallas_call(
        _mxu_main_body,
        grid=(rows // br,),
        in_specs=[
            pl.BlockSpec((_D, _D), lambda i: (0, 0)),
            pl.BlockSpec((br, _D), lambda i: (i, 0)),
        ],
        out_specs=pl.BlockSpec((br, _D), lambda i: (i, 0)),
        out_shape=jax.ShapeDtypeStruct((rows, _D), xf.dtype),
    )(m, xf)


def _tc_apply(xf, local_angles, ent_angles, br=512):
    rows = xf.shape[0]
    theta = _theta_table(local_angles, ent_angles)
    return pl.pallas_call(
        _tc_body,
        grid=(rows // br,),
        in_specs=[
            pl.BlockSpec((_NSTEP, _D), lambda i: (0, 0)),
            pl.BlockSpec((br, _D), lambda i: (i, 0)),
        ],
        out_specs=pl.BlockSpec((br, _D), lambda i: (i, 0)),
        out_shape=jax.ShapeDtypeStruct((rows, _D), xf.dtype),
    )(theta, xf)


def _sc_step(src, dst, k, ctab, stab, lane):
    p = 1 if k % 3 == 1 else 0

    def chunk(q, carry):
        lo = 2 * (q * 16 + lane) + p
        hi = (lo + 1) & (_D - 1)
        cvec = ctab[pl.ds(k * 512 + q * 16, 16)]
        svec = stab[pl.ds(k * 512 + q * 16, 16)]
        for r in range(_G):
            ilo = r * _D + lo
            ihi = r * _D + hi
            vlo = plsc.load_gather(src, [ilo])
            vhi = plsc.load_gather(src, [ihi])
            plsc.store_scatter(dst, [ilo], cvec * vlo + svec * vhi)
            plsc.store_scatter(dst, [ihi], cvec * vhi - svec * vlo)
        return carry

    lax.fori_loop(0, 32, chunk, 0)


def _sc_body_async(x_hbm, c_hbm, s_hbm, o_hbm,
                   a0, a1, bscr, cscr, ob0, ob1, ctab, stab,
                   si0, si1, so0, so1):
    wid = lax.axis_index("s") * 2 + lax.axis_index("c")
    pltpu.sync_copy(c_hbm, ctab)
    pltpu.sync_copy(s_hbm, stab)
    nelem = x_hbm.shape[0]
    epw = nelem // _NW
    gsz = _G * _D
    ng = epw // gsz
    base = wid * epw
    lane = lax.iota(jnp.int32, 16)
    pltpu.async_copy(x_hbm.at[pl.ds(base, gsz)], a0, si0)
    pltpu.async_copy(x_hbm.at[pl.ds(base + gsz, gsz)], a1, si1)

    def pair(h, carry):
        for phase in range(2):
            ab, ob, si, so = ((a0, ob0, si0, so0) if phase == 0
                              else (a1, ob1, si1, so1))
            g = 2 * h + phase
            e0 = base + g * gsz
            pltpu.make_async_copy(x_hbm.at[pl.ds(e0, gsz)], ab, si).wait()

            @pl.when(g >= 2)
            def _():
                pltpu.make_async_copy(
                    ob, o_hbm.at[pl.ds(e0 - 2 * gsz, gsz)], so).wait()

            _sc_step(ab, bscr, 0, ctab, stab, lane)

            @pl.when(g + 2 < ng)
            def _():
                pltpu.async_copy(x_hbm.at[pl.ds(e0 + 2 * gsz, gsz)], ab, si)

            for k, (src, dst) in enumerate(
                    [(bscr, cscr), (cscr, bscr), (bscr, cscr),
                     (cscr, bscr), (bscr, ob)], start=1):
                _sc_step(src, dst, k, ctab, stab, lane)
            pltpu.async_copy(ob, o_hbm.at[pl.ds(e0, gsz)], so)
        return carry

    lax.fori_loop(0, ng // 2, pair, 0)
    pltpu.make_async_copy(
        ob0, o_hbm.at[pl.ds(base + (ng - 2) * gsz, gsz)], so0).wait()
    pltpu.make_async_copy(
        ob1, o_hbm.at[pl.ds(base + (ng - 1) * gsz, gsz)], so1).wait()


def _sc_apply_async(xf, ctab, stab):
    n = xf.shape[0] * xf.shape[1]
    mesh = plsc.VectorSubcoreMesh(core_axis_name="c", subcore_axis_name="s")
    out = pl.kernel(
        _sc_body_async,
        out_type=jax.ShapeDtypeStruct((n,), xf.dtype),
        mesh=mesh,
        scratch_types=[
            pltpu.VMEM((_G * _D,), jnp.float32),
            pltpu.VMEM((_G * _D,), jnp.float32),
            pltpu.VMEM((_G * _D,), jnp.float32),
            pltpu.VMEM((_G * _D,), jnp.float32),
            pltpu.VMEM((_G * _D,), jnp.float32),
            pltpu.VMEM((_G * _D,), jnp.float32),
            pltpu.VMEM((_NSTEP * 512,), jnp.float32),
            pltpu.VMEM((_NSTEP * 512,), jnp.float32),
            pltpu.SemaphoreType.DMA,
            pltpu.SemaphoreType.DMA,
            pltpu.SemaphoreType.DMA,
            pltpu.SemaphoreType.DMA,
        ],
        compiler_params=pltpu.CompilerParams(needs_layout_passes=False),
    )(xf.reshape(n), ctab.reshape(-1), stab.reshape(-1))
    return out.reshape(xf.shape)


def _sc_body(x_hbm, c_hbm, s_hbm, o_hbm, buf, bufb, ctab, stab):
    wid = lax.axis_index("s") * 2 + lax.axis_index("c")
    pltpu.sync_copy(c_hbm, ctab)
    pltpu.sync_copy(s_hbm, stab)
    nelem = x_hbm.shape[0]
    epw = nelem // _NW
    ngroups = epw // (_G * _D)
    base = wid * epw
    lane = lax.iota(jnp.int32, 16)

    def group(g, carry):
        e0 = base + g * _G * _D
        pltpu.sync_copy(x_hbm.at[pl.ds(e0, _G * _D)], buf)
        for k in range(_NSTEP):
            p = 1 if k % 3 == 1 else 0
            src = buf if k % 2 == 0 else bufb
            dst = bufb if k % 2 == 0 else buf

            def chunk(q, carry2):
                lo = 2 * (q * 16 + lane) + p
                hi = (lo + 1) & (_D - 1)
                cvec = ctab[pl.ds(k * 512 + q * 16, 16)]
                svec = stab[pl.ds(k * 512 + q * 16, 16)]

                for r in range(_G):
                    ilo = r * _D + lo
                    ihi = r * _D + hi
                    vlo = plsc.load_gather(src, [ilo])
                    vhi = plsc.load_gather(src, [ihi])
                    nlo = cvec * vlo + svec * vhi
                    nhi = cvec * vhi - svec * vlo
                    plsc.store_scatter(dst, [ilo], nlo)
                    plsc.store_scatter(dst, [ihi], nhi)
                return carry2

            lax.fori_loop(0, 32, chunk, 0)
        pltpu.sync_copy(buf, o_hbm.at[pl.ds(e0, _G * _D)])
        return carry

    lax.fori_loop(0, ngroups, group, 0)


def _sc_apply(xf, ctab, stab):
    n = xf.shape[0] * xf.shape[1]
    mesh = plsc.VectorSubcoreMesh(core_axis_name="c", subcore_axis_name="s")
    out = pl.kernel(
        _sc_body,
        out_type=jax.ShapeDtypeStruct((n,), xf.dtype),
        mesh=mesh,
        scratch_types=[
            pltpu.VMEM((_G * _D,), jnp.float32),
            pltpu.VMEM((_G * _D,), jnp.float32),
            pltpu.VMEM((_NSTEP * 512,), jnp.float32),
            pltpu.VMEM((_NSTEP * 512,), jnp.float32),
        ],
        compiler_params=pltpu.CompilerParams(needs_layout_passes=False),
    )(xf.reshape(n), ctab.reshape(-1), stab.reshape(-1))
    return out.reshape(xf.shape)


def kernel(x, local_angles, ent_angles):
    b, s, d = x.shape
    xf = x.reshape(b * s, d)
    ctab, stab = _pair_trig_tables(local_angles, ent_angles)
    out = _sc_apply_async(xf, ctab, stab)
    return out.reshape(b, s, d)


# speedup vs baseline: 11.4068x; 11.4068x over previous
import jax
import jax.numpy as jnp
import numpy as np
from jax import lax
from jax.experimental import pallas as pl
from jax.experimental.pallas import tpu as pltpu
from jax.experimental.pallas import tpu_sc as plsc

_D = 1024
_NL = 2
_NSTEP = 3 * _NL
_NW = 32
_G = 16


def _trig_body(t_ref, c_ref, s_ref):
    t = t_ref[...]
    c_ref[...] = jnp.cos(t)
    s_ref[...] = jnp.sin(t)


def _pair_trig_tables(local_angles, ent_angles):
    rows = []
    for l in range(_NL):
        rows.append(local_angles[l, 0::2])
        rows.append(local_angles[l, 1::2])
        rows.append(ent_angles[l])
    theta = jnp.stack(rows)
    return pl.pallas_call(
        _trig_body,
        out_shape=(
            jax.ShapeDtypeStruct(theta.shape, theta.dtype),
            jax.ShapeDtypeStruct(theta.shape, theta.dtype),
        ),
    )(theta)


def _tc_body(theta_ref, x_ref, o_ref):
    v = x_ref[...]
    lane = lax.broadcasted_iota(jnp.int32, (1, _D), 1)
    parity = lane % 2
    for k in range(_NSTEP):
        p = 1 if (k % 3 == 1) else 0
        t = theta_ref[k, :].reshape(1, _D)
        c = jnp.cos(t)
        s = jnp.sin(t)
        is_lo = parity == p
        s_signed = jnp.where(is_lo, s, -s)
        partner = jnp.where(is_lo, jnp.roll(v, -1, axis=1), jnp.roll(v, 1, axis=1))
        v = c * v + s_signed * partner
    o_ref[...] = v


def _theta_table(local_angles, ent_angles):
    evenm = jnp.asarray((np.arange(_D) % 2) == 0)
    rows = []
    for l in range(_NL):
        a = local_angles[l]
        rows.append(jnp.where(evenm, a, jnp.roll(a, 1)))
        rows.append(jnp.where(~evenm, a, jnp.roll(a, 1)))
        rows.append(jnp.repeat(ent_angles[l], 2))
    return jnp.stack(rows)


def _band_body(theta_ref, w_ref):
    lane = lax.broadcasted_iota(jnp.int32, (1, _D), 1)
    parity = lane % 2
    row = lax.broadcasted_iota(jnp.int32, (16, _D), 0)
    b_mat = jnp.where(row == 8, 1.0, 0.0)
    for k in range(_NSTEP):
        p = 1 if (k % 3 == 1) else 0
        t = theta_ref[k, :].reshape(1, _D)
        c = jnp.cos(t)
        s = jnp.sin(t)
        is_lo = parity == p
        a = c
        b = jnp.where(is_lo, s, 0.0)
        d = jnp.where(is_lo, 0.0, -s)
        b_mat = (
            a * b_mat
            + b * jnp.roll(jnp.roll(b_mat, 1, axis=0), -1, axis=1)
            + d * jnp.roll(jnp.roll(b_mat, -1, axis=0), 1, axis=1)
        )
    w_ref[...] = b_mat


def _banded_main_body(w_ref, x_ref, o_ref):
    v = x_ref[...]
    acc = w_ref[8, :].reshape(1, _D) * v
    for r in list(range(2, 8)) + list(range(9, 15)):
        acc += w_ref[r, :].reshape(1, _D) * jnp.roll(v, 8 - r, axis=1)
    o_ref[...] = acc


def _tc_apply_banded(xf, local_angles, ent_angles, br=512):
    rows = xf.shape[0]
    theta = _theta_table(local_angles, ent_angles)
    w = pl.pallas_call(
        _band_body,
        out_shape=jax.ShapeDtypeStruct((16, _D), jnp.float32),
    )(theta)
    return pl.pallas_call(
        _banded_main_body,
        grid=(rows // br,),
        in_specs=[
            pl.BlockSpec((16, _D), lambda i: (0, 0)),
            pl.BlockSpec((br, _D), lambda i: (i, 0)),
        ],
        out_specs=pl.BlockSpec((br, _D), lambda i: (i, 0)),
        out_shape=jax.ShapeDtypeStruct((rows, _D), xf.dtype),
    )(w, xf)


def _mat_body(w_ref, m_ref):
    ri = lax.broadcasted_iota(jnp.int32, (_D, _D), 0)
    ci = lax.broadcasted_iota(jnp.int32, (_D, _D), 1)
    acc = jnp.zeros((_D, _D), jnp.float32)
    for r in range(2, 15):
        wcol = w_ref[r, :].reshape(1, _D)
        acc = jnp.where((ri - ci - (r - 8)) % _D == 0, wcol, acc)
    m_ref[...] = acc


def _mxu_main_body(m_ref, x_ref, o_ref):
    o_ref[...] = jax.lax.dot_general(
        x_ref[...], m_ref[...],
        (((1,), (0,)), ((), ())),
        preferred_element_type=jnp.float32,
    )


def _tc_apply_mxu(xf, local_angles, ent_angles, br=512):
    rows = xf.shape[0]
    theta = _theta_table(local_angles, ent_angles)
    w = pl.pallas_call(
        _band_body,
        out_shape=jax.ShapeDtypeStruct((16, _D), jnp.float32),
    )(theta)
    m = pl.pallas_call(
        _mat_body,
        out_shape=jax.ShapeDtypeStruct((_D, _D), jnp.float32),
    )(w)
    return pl.pallas_call(
        _mxu_main_body,
        grid=(rows // br,),
        in_specs=[
            pl.BlockSpec((_D, _D), lambda i: (0, 0)),
            pl.BlockSpec((br, _D), lambda i: (i, 0)),
        ],
        out_specs=pl.BlockSpec((br, _D), lambda i: (i, 0)),
        out_shape=jax.ShapeDtypeStruct((rows, _D), xf.dtype),
    )(m, xf)


def _tc_apply(xf, local_angles, ent_angles, br=512):
    rows = xf.shape[0]
    theta = _theta_table(local_angles, ent_angles)
    return pl.pallas_call(
        _tc_body,
        grid=(rows // br,),
        in_specs=[
            pl.BlockSpec((_NSTEP, _D), lambda i: (0, 0)),
            pl.BlockSpec((br, _D), lambda i: (i, 0)),
        ],
        out_specs=pl.BlockSpec((br, _D), lambda i: (i, 0)),
        out_shape=jax.ShapeDtypeStruct((rows, _D), xf.dtype),
    )(theta, xf)


def _sc_step(src, dst, k, ctab, stab, lane):
    p = 1 if k % 3 == 1 else 0

    def chunk(q, carry):
        lo = 2 * (q * 16 + lane) + p
        hi = (lo + 1) & (_D - 1)
        cvec = ctab[pl.ds(k * 512 + q * 16, 16)]
        svec = stab[pl.ds(k * 512 + q * 16, 16)]
        for r in range(_G):
            ilo = r * _D + lo
            ihi = r * _D + hi
            vlo = plsc.load_gather(src, [ilo])
            vhi = plsc.load_gather(src, [ihi])
            plsc.store_scatter(dst, [ilo], cvec * vlo + svec * vhi)
            plsc.store_scatter(dst, [ihi], cvec * vhi - svec * vlo)
        return carry

    lax.fori_loop(0, 32, chunk, 0)


def _sc_body_async(x_hbm, c_hbm, s_hbm, o_hbm,
                   a0, a1, bscr, cscr, ob0, ob1, ctab, stab,
                   si0, si1, so0, so1):
    wid = lax.axis_index("s") * 2 + lax.axis_index("c")
    pltpu.sync_copy(c_hbm, ctab)
    pltpu.sync_copy(s_hbm, stab)
    nelem = x_hbm.shape[0]
    epw = nelem // _NW
    gsz = _G * _D
    ng = epw // gsz
    base = wid * epw
    lane = lax.iota(jnp.int32, 16)
    pltpu.async_copy(x_hbm.at[pl.ds(base, gsz)], a0, si0)
    pltpu.async_copy(x_hbm.at[pl.ds(base + gsz, gsz)], a1, si1)

    def pair(h, carry):
        for phase in range(2):
            ab, ob, si, so = ((a0, ob0, si0, so0) if phase == 0
                              else (a1, ob1, si1, so1))
            g = 2 * h + phase
            e0 = base + g * gsz
            pltpu.make_async_copy(x_hbm.at[pl.ds(e0, gsz)], ab, si).wait()

            @pl.when(g >= 2)
            def _():
                pltpu.make_async_copy(
                    ob, o_hbm.at[pl.ds(e0 - 2 * gsz, gsz)], so).wait()

            _sc_step(ab, bscr, 0, ctab, stab, lane)

            @pl.when(g + 2 < ng)
            def _():
                pltpu.async_copy(x_hbm.at[pl.ds(e0 + 2 * gsz, gsz)], ab, si)

            for k, (src, dst) in enumerate(
                    [(bscr, cscr), (cscr, bscr), (bscr, cscr),
                     (cscr, bscr), (bscr, ob)], start=1):
                _sc_step(src, dst, k, ctab, stab, lane)
            pltpu.async_copy(ob, o_hbm.at[pl.ds(e0, gsz)], so)
        return carry

    lax.fori_loop(0, ng // 2, pair, 0)
    pltpu.make_async_copy(
        ob0, o_hbm.at[pl.ds(base + (ng - 2) * gsz, gsz)], so0).wait()
    pltpu.make_async_copy(
        ob1, o_hbm.at[pl.ds(base + (ng - 1) * gsz, gsz)], so1).wait()


def _sc_apply_async(xf, ctab, stab):
    n = xf.shape[0] * xf.shape[1]
    mesh = plsc.VectorSubcoreMesh(core_axis_name="c", subcore_axis_name="s")
    out = pl.kernel(
        _sc_body_async,
        out_type=jax.ShapeDtypeStruct((n,), xf.dtype),
        mesh=mesh,
        scratch_types=[
            pltpu.VMEM((_G * _D,), jnp.float32),
            pltpu.VMEM((_G * _D,), jnp.float32),
            pltpu.VMEM((_G * _D,), jnp.float32),
            pltpu.VMEM((_G * _D,), jnp.float32),
            pltpu.VMEM((_G * _D,), jnp.float32),
            pltpu.VMEM((_G * _D,), jnp.float32),
            pltpu.VMEM((_NSTEP * 512,), jnp.float32),
            pltpu.VMEM((_NSTEP * 512,), jnp.float32),
            pltpu.SemaphoreType.DMA,
            pltpu.SemaphoreType.DMA,
            pltpu.SemaphoreType.DMA,
            pltpu.SemaphoreType.DMA,
        ],
        compiler_params=pltpu.CompilerParams(needs_layout_passes=False),
    )(xf.reshape(n), ctab.reshape(-1), stab.reshape(-1))
    return out.reshape(xf.shape)


def _sc_body(x_hbm, c_hbm, s_hbm, o_hbm, buf, bufb, ctab, stab):
    wid = lax.axis_index("s") * 2 + lax.axis_index("c")
    pltpu.sync_copy(c_hbm, ctab)
    pltpu.sync_copy(s_hbm, stab)
    nelem = x_hbm.shape[0]
    epw = nelem // _NW
    ngroups = epw // (_G * _D)
    base = wid * epw
    lane = lax.iota(jnp.int32, 16)

    def group(g, carry):
        e0 = base + g * _G * _D
        pltpu.sync_copy(x_hbm.at[pl.ds(e0, _G * _D)], buf)
        for k in range(_NSTEP):
            p = 1 if k % 3 == 1 else 0
            src = buf if k % 2 == 0 else bufb
            dst = bufb if k % 2 == 0 else buf

            def chunk(q, carry2):
                lo = 2 * (q * 16 + lane) + p
                hi = (lo + 1) & (_D - 1)
                cvec = ctab[pl.ds(k * 512 + q * 16, 16)]
                svec = stab[pl.ds(k * 512 + q * 16, 16)]

                for r in range(_G):
                    ilo = r * _D + lo
                    ihi = r * _D + hi
                    vlo = plsc.load_gather(src, [ilo])
                    vhi = plsc.load_gather(src, [ihi])
                    nlo = cvec * vlo + svec * vhi
                    nhi = cvec * vhi - svec * vlo
                    plsc.store_scatter(dst, [ilo], nlo)
                    plsc.store_scatter(dst, [ihi], nhi)
                return carry2

            lax.fori_loop(0, 32, chunk, 0)
        pltpu.sync_copy(buf, o_hbm.at[pl.ds(e0, _G * _D)])
        return carry

    lax.fori_loop(0, ngroups, group, 0)


def _sc_apply(xf, ctab, stab):
    n = xf.shape[0] * xf.shape[1]
    mesh = plsc.VectorSubcoreMesh(core_axis_name="c", subcore_axis_name="s")
    out = pl.kernel(
        _sc_body,
        out_type=jax.ShapeDtypeStruct((n,), xf.dtype),
        mesh=mesh,
        scratch_types=[
            pltpu.VMEM((_G * _D,), jnp.float32),
            pltpu.VMEM((_G * _D,), jnp.float32),
            pltpu.VMEM((_NSTEP * 512,), jnp.float32),
            pltpu.VMEM((_NSTEP * 512,), jnp.float32),
        ],
        compiler_params=pltpu.CompilerParams(needs_layout_passes=False),
    )(xf.reshape(n), ctab.reshape(-1), stab.reshape(-1))
    return out.reshape(xf.shape)


_SC_ROWS = 8192


def _hybrid_apply(x, local_angles, ent_angles):
    b, s, d = x.shape
    xf = x.reshape(b * s, d)
    ctab, stab = _pair_trig_tables(local_angles, ent_angles)
    out_sc = _sc_apply_async(xf[:_SC_ROWS], ctab, stab)
    out_tc = _tc_apply_mxu(xf[_SC_ROWS:], local_angles, ent_angles)
    return jnp.concatenate([out_sc, out_tc], axis=0).reshape(b, s, d)


def kernel(x, local_angles, ent_angles):
    b, s, d = x.shape
    xf = x.reshape(b * s, d)
    out = _tc_apply_mxu(xf, local_angles, ent_angles)
    return out.reshape(b, s, d)
